# Initial kernel scaffold; baseline (speedup 1.0000x reference)
import jax
import jax.numpy as jnp
from jax.experimental import pallas as pl


def kernel(x, Wr, We, be):
    raise NotImplementedError("write your pallas kernel here")


# speedup vs baseline: 1.3669x; 1.3669x over previous
import functools

import jax
import jax.numpy as jnp
from jax.experimental import pallas as pl
from jax.experimental.pallas import tpu as pltpu

B, S, D, E, K, DO = 2, 2048, 1024, 8, 2, 1024
N = B * S
T = 512


def _moe_body(x_ref, wr_ref, we_ref, be_ref, out_ref,
              acc_ref, a1_ref, a2_ref, w1_ref, w2_ref):
    e = pl.program_id(1)

    @pl.when(e == 0)
    def _router():
        logits = jnp.dot(x_ref[...], wr_ref[...],
                         preferred_element_type=jnp.float32)
        iota = jax.lax.broadcasted_iota(jnp.int32, (T, E), 1)
        m1 = jnp.max(logits, axis=-1, keepdims=True)
        a1 = jnp.argmax(logits, axis=-1)[:, None]
        masked = jnp.where(iota == a1, -jnp.inf, logits)
        m2 = jnp.max(masked, axis=-1, keepdims=True)
        a2 = jnp.argmax(masked, axis=-1)[:, None]
        z = jnp.exp(m2 - m1)
        a1_ref[...] = a1
        a2_ref[...] = a2
        w1_ref[...] = 1.0 / (1.0 + z)
        w2_ref[...] = z / (1.0 + z)
        acc_ref[...] = jnp.zeros_like(acc_ref)

    expert_out = jax.nn.relu(
        jnp.dot(x_ref[...], we_ref[0],
                preferred_element_type=jnp.float32) + be_ref[0])
    gate = (jnp.where(a1_ref[...] == e, w1_ref[...], 0.0)
            + jnp.where(a2_ref[...] == e, w2_ref[...], 0.0))
    acc_ref[...] += gate * expert_out

    @pl.when(e == E - 1)
    def _flush():
        out_ref[...] = acc_ref[...]


@functools.partial(jax.jit)
def _moe(x2, Wr, We, be):
    grid = (N // T, E)
    return pl.pallas_call(
        _moe_body,
        grid=grid,
        in_specs=[
            pl.BlockSpec((T, D), lambda t, e: (t, 0)),
            pl.BlockSpec((D, E), lambda t, e: (0, 0)),
            pl.BlockSpec((1, D, DO), lambda t, e: (e, 0, 0)),
            pl.BlockSpec((1, 1, DO), lambda t, e: (e, 0, 0)),
        ],
        out_specs=pl.BlockSpec((T, DO), lambda t, e: (t, 0)),
        out_shape=jax.ShapeDtypeStruct((N, DO), jnp.float32),
        scratch_shapes=[
            pltpu.VMEM((T, DO), jnp.float32),
            pltpu.VMEM((T, 1), jnp.int32),
            pltpu.VMEM((T, 1), jnp.int32),
            pltpu.VMEM((T, 1), jnp.float32),
            pltpu.VMEM((T, 1), jnp.float32),
        ],
    )(x2, Wr, We, be.reshape(E, 1, DO))


def kernel(x, Wr, We, be):
    out = _moe(x.reshape(N, D), Wr, We, be)
    return out.reshape(B, S, DO)


# speedup vs baseline: 1.6347x; 1.1959x over previous
import functools

import jax
import jax.numpy as jnp
from jax.experimental import pallas as pl
from jax.experimental.pallas import tpu as pltpu

B, S, D, E, K, DO = 2, 2048, 1024, 8, 2, 1024
N = B * S
T = 1024


def _moe_body(x_ref, wr_ref, we_ref, be_ref, out_ref,
              acc_ref, a1_ref, a2_ref, w1_ref, w2_ref):
    e = pl.program_id(1)

    @pl.when(e == 0)
    def _router():
        logits = jnp.dot(x_ref[...], wr_ref[...],
                         preferred_element_type=jnp.float32)
        iota = jax.lax.broadcasted_iota(jnp.int32, (T, E), 1)
        m1 = jnp.max(logits, axis=-1, keepdims=True)
        a1 = jnp.argmax(logits, axis=-1)[:, None]
        masked = jnp.where(iota == a1, -jnp.inf, logits)
        m2 = jnp.max(masked, axis=-1, keepdims=True)
        a2 = jnp.argmax(masked, axis=-1)[:, None]
        z = jnp.exp(m2 - m1)
        a1_ref[...] = a1
        a2_ref[...] = a2
        w1_ref[...] = 1.0 / (1.0 + z)
        w2_ref[...] = z / (1.0 + z)
        acc_ref[...] = jnp.zeros_like(acc_ref)

    expert_out = jax.nn.relu(
        jnp.dot(x_ref[...].astype(jnp.bfloat16), we_ref[0],
                preferred_element_type=jnp.float32) + be_ref[0])
    gate = (jnp.where(a1_ref[...] == e, w1_ref[...], 0.0)
            + jnp.where(a2_ref[...] == e, w2_ref[...], 0.0))
    acc_ref[...] += gate * expert_out

    @pl.when(e == E - 1)
    def _flush():
        out_ref[...] = acc_ref[...]


@functools.partial(jax.jit)
def _moe(x2, Wr, We, be):
    grid = (N // T, E)
    return pl.pallas_call(
        _moe_body,
        grid=grid,
        in_specs=[
            pl.BlockSpec((T, D), lambda t, e: (t, 0)),
            pl.BlockSpec((D, E), lambda t, e: (0, 0)),
            pl.BlockSpec((1, D, DO), lambda t, e: (e, 0, 0)),
            pl.BlockSpec((1, 1, DO), lambda t, e: (e, 0, 0)),
        ],
        out_specs=pl.BlockSpec((T, DO), lambda t, e: (t, 0)),
        out_shape=jax.ShapeDtypeStruct((N, DO), jnp.float32),
        scratch_shapes=[
            pltpu.VMEM((T, DO), jnp.float32),
            pltpu.VMEM((T, 1), jnp.int32),
            pltpu.VMEM((T, 1), jnp.int32),
            pltpu.VMEM((T, 1), jnp.float32),
            pltpu.VMEM((T, 1), jnp.float32),
        ],
    )(x2, Wr, We.astype(jnp.bfloat16), be.reshape(E, 1, DO))


def kernel(x, Wr, We, be):
    out = _moe(x.reshape(N, D), Wr, We, be)
    return out.reshape(B, S, DO)
